# baseline (speedup 1.0000x reference)
import jax
import jax.numpy as jnp
from jax import lax
from jax.experimental import pallas as pl
from jax.experimental.pallas import tpu as pltpu
from jax.experimental.pallas import tpu_sc as plsc

_B, _R, _D_IN, _D_OUT = 4096, 200, 128, 64
_SPLIT = 3072
_NC, _NS = 2, 16
_NW = _NC * _NS
_SC_B = _B - _SPLIT
_BPW = _SC_B // _NW
_NG = _D_OUT // 16
_BM = 32
_BS = 32


def _sc_body(x_hbm, idx_hbm, out_hbm, idx_v, in_bufs, out_bufs, in_sems, out_sems):
    wid = lax.axis_index("s") * _NC + lax.axis_index("c")
    base = _SPLIT + wid * _BPW
    obase = wid * _BPW
    pltpu.sync_copy(idx_hbm, idx_v)
    idxv = [idx_v[pl.ds(16 * k, 16)] for k in range(_NG)]

    def compute(in_buf, out_buf):
        @plsc.parallel_loop(0, _R, unroll=8)
        def _(r):
            row = jnp.full((16,), r, jnp.int32)
            for k in range(_NG):
                v = plsc.load_gather(in_buf, [row, idxv[k]])
                out_buf[r, pl.ds(16 * k, 16)] = v

    pltpu.async_copy(x_hbm.at[base], in_bufs[0], in_sems[0])

    def pair_body(i, carry):
        for b in range(2):
            c = 2 * i + b
            nxt = c + 1

            @pl.when(nxt < _BPW)
            def _():
                pltpu.async_copy(x_hbm.at[base + nxt], in_bufs[1 - b], in_sems[1 - b])
            pltpu.make_async_copy(x_hbm.at[base + c], in_bufs[b], in_sems[b]).wait()

            @pl.when(c >= 2)
            def _():
                pltpu.make_async_copy(
                    out_bufs[b], out_hbm.at[obase + c - 2], out_sems[b]).wait()
            compute(in_bufs[b], out_bufs[b])
            pltpu.async_copy(out_bufs[b], out_hbm.at[obase + c], out_sems[b])
        return carry

    lax.fori_loop(0, _BPW // 2, pair_body, 0)
    pltpu.make_async_copy(out_bufs[0], out_hbm.at[obase + _BPW - 2], out_sems[0]).wait()
    pltpu.make_async_copy(out_bufs[1], out_hbm.at[obase + _BPW - 1], out_sems[1]).wait()


def _sc_tail(x, indices):
    sc_call = pl.kernel(
        _sc_body,
        out_type=jax.ShapeDtypeStruct((_SC_B, _R, _D_OUT), jnp.float32),
        mesh=plsc.VectorSubcoreMesh(core_axis_name="c", subcore_axis_name="s"),
        scratch_types=[
            pltpu.VMEM((_D_OUT,), jnp.int32),
            [pltpu.VMEM((_R, _D_IN), jnp.float32) for _ in range(2)],
            [pltpu.VMEM((_R, _D_OUT), jnp.float32) for _ in range(2)],
            [pltpu.SemaphoreType.DMA for _ in range(2)],
            [pltpu.SemaphoreType.DMA for _ in range(2)],
        ],
        compiler_params=pltpu.CompilerParams(needs_layout_passes=False),
    )
    return sc_call(x, indices)


def _tc_body(x_ref, idx_ref, out_ref):
    x = x_ref[...]
    idx = jnp.broadcast_to(idx_ref[...], (_BM, _R, _D_OUT))
    out_ref[...] = jnp.take_along_axis(x, idx, axis=-1)


def _tc_head(x, indices):
    grid = (_SPLIT // _BM,)
    return pl.pallas_call(
        _tc_body,
        grid=grid,
        in_specs=[pl.BlockSpec((_BM, _R, _D_IN), lambda i: (i, 0, 0)),
                  pl.BlockSpec((1, 1, _D_OUT), lambda i: (0, 0, 0))],
        out_specs=pl.BlockSpec((_BM, _R, _D_OUT), lambda i: (i, 0, 0)),
        out_shape=jax.ShapeDtypeStruct((_B, _R, _D_OUT), jnp.float32),
    )(x, indices.reshape(1, 1, _D_OUT))


def _stitch_body(sc_ref, tcfull_ref, out_ref):
    del tcfull_ref
    out_ref[...] = sc_ref[...]


def _stitch(sc_out, tc_out):
    grid = (_SC_B // _BS,)
    return pl.pallas_call(
        _stitch_body,
        grid=grid,
        in_specs=[pl.BlockSpec((_BS, _R, _D_OUT), lambda i: (i, 0, 0)),
                  pl.BlockSpec(memory_space=pltpu.HBM)],
        out_specs=pl.BlockSpec((_BS, _R, _D_OUT),
                               lambda i: (_SPLIT // _BS + i, 0, 0)),
        out_shape=jax.ShapeDtypeStruct((_B, _R, _D_OUT), jnp.float32),
        input_output_aliases={1: 0},
    )(sc_out, tc_out)


def kernel(x, indices):
    sc_out = _sc_tail(x, indices)
    tc_out = _tc_head(x, indices)
    return _stitch(sc_out, tc_out)


# speedup vs baseline: 1.0042x; 1.0042x over previous
import jax
import jax.numpy as jnp
from jax import lax
from jax.experimental import pallas as pl
from jax.experimental.pallas import tpu as pltpu
from jax.experimental.pallas import tpu_sc as plsc

_B, _R, _D_IN, _D_OUT = 4096, 200, 128, 64
_SPLIT = 3072
_NC, _NS = 2, 16
_NW = _NC * _NS
_SC_B = _B - _SPLIT
_BPW = _SC_B // _NW
_NG = _D_OUT // 16
_BM = 32
_BS = 32


def _sc_body(x_hbm, idx_hbm, out_hbm, idx_v, in_bufs, out_bufs, in_sems, out_sems):
    wid = lax.axis_index("s") * _NC + lax.axis_index("c")
    base = _SPLIT + wid * _BPW
    obase = wid * _BPW
    pltpu.sync_copy(idx_hbm, idx_v)
    idxv = [idx_v[pl.ds(16 * k, 16)] for k in range(_NG)]

    def compute(in_buf, out_buf):
        @plsc.parallel_loop(0, _R, unroll=8)
        def _(r):
            row = jnp.full((16,), r, jnp.int32)
            for k in range(_NG):
                v = plsc.load_gather(in_buf, [row, idxv[k]])
                out_buf[r, pl.ds(16 * k, 16)] = v

    pltpu.async_copy(x_hbm.at[base], in_bufs[0], in_sems[0])

    def pair_body(i, carry):
        for b in range(2):
            c = 2 * i + b
            nxt = c + 1

            @pl.when(nxt < _BPW)
            def _():
                pltpu.async_copy(x_hbm.at[base + nxt], in_bufs[1 - b], in_sems[1 - b])
            pltpu.make_async_copy(x_hbm.at[base + c], in_bufs[b], in_sems[b]).wait()

            @pl.when(c >= 2)
            def _():
                pltpu.make_async_copy(
                    out_bufs[b], out_hbm.at[obase + c - 2], out_sems[b]).wait()
            compute(in_bufs[b], out_bufs[b])
            pltpu.async_copy(out_bufs[b], out_hbm.at[obase + c], out_sems[b])
        return carry

    lax.fori_loop(0, _BPW // 2, pair_body, 0)
    pltpu.make_async_copy(out_bufs[0], out_hbm.at[obase + _BPW - 2], out_sems[0]).wait()
    pltpu.make_async_copy(out_bufs[1], out_hbm.at[obase + _BPW - 1], out_sems[1]).wait()


def _sc_tail(x, indices):
    sc_call = pl.kernel(
        _sc_body,
        out_type=jax.ShapeDtypeStruct((_SC_B, _R, _D_OUT), jnp.float32),
        mesh=plsc.VectorSubcoreMesh(core_axis_name="c", subcore_axis_name="s"),
        scratch_types=[
            pltpu.VMEM((_D_OUT,), jnp.int32),
            [pltpu.VMEM((_R, _D_IN), jnp.float32) for _ in range(2)],
            [pltpu.VMEM((_R, _D_OUT), jnp.float32) for _ in range(2)],
            [pltpu.SemaphoreType.DMA for _ in range(2)],
            [pltpu.SemaphoreType.DMA for _ in range(2)],
        ],
        compiler_params=pltpu.CompilerParams(needs_layout_passes=False),
    )
    return sc_call(x, indices)


def _tc_body(x_ref, idx_ref, out_ref):
    x = x_ref[...]
    idx = jnp.broadcast_to(idx_ref[...], (_BM, _R, _D_OUT))
    out_ref[...] = jnp.take_along_axis(x, idx, axis=-1)


def _tc_head(x, indices):
    grid = (_SPLIT // _BM,)
    return pl.pallas_call(
        _tc_body,
        grid=grid,
        in_specs=[pl.BlockSpec((_BM, _R, _D_IN), lambda i: (i, 0, 0)),
                  pl.BlockSpec((1, 1, _D_OUT), lambda i: (0, 0, 0))],
        out_specs=pl.BlockSpec((_BM, _R, _D_OUT), lambda i: (i, 0, 0)),
        out_shape=jax.ShapeDtypeStruct((_B, _R, _D_OUT), jnp.float32),
    )(x, indices.reshape(1, 1, _D_OUT))


def _stitch_body(sc_ref, tcfull_ref, out_ref):
    del tcfull_ref
    out_ref[...] = sc_ref[...]


def _stitch(sc_out, tc_out):
    grid = (_SC_B // _BS,)
    return pl.pallas_call(
        _stitch_body,
        grid=grid,
        in_specs=[pl.BlockSpec((_BS, _R, _D_OUT), lambda i: (i, 0, 0)),
                  pl.BlockSpec(memory_space=pl.ANY)],
        out_specs=pl.BlockSpec((_BS, _R, _D_OUT),
                               lambda i: (_SPLIT // _BS + i, 0, 0)),
        out_shape=jax.ShapeDtypeStruct((_B, _R, _D_OUT), jnp.float32),
        input_output_aliases={1: 0},
    )(sc_out, tc_out)


def kernel(x, indices):
    sc_out = _sc_tail(x, indices)
    tc_out = _tc_head(x, indices)
    return _stitch(sc_out, tc_out)


# speedup vs baseline: 1.2050x; 1.2000x over previous
import jax
import jax.numpy as jnp
from jax import lax
from jax.experimental import pallas as pl
from jax.experimental.pallas import tpu as pltpu
from jax.experimental.pallas import tpu_sc as plsc

_B, _R, _D_IN, _D_OUT = 4096, 200, 128, 64
_SPLIT = 3072
_NC, _NS = 2, 16
_NW = _NC * _NS
_SC_B = _B - _SPLIT
_BPW = _SC_B // _NW
_NG = _D_OUT // 16
_BM = 32
_BS = 32


def _sc_body(x_hbm, idx_hbm, out_hbm, idx_v, in_bufs, out_bufs, in_sems, out_sems):
    wid = lax.axis_index("s") * _NC + lax.axis_index("c")
    base = _SPLIT + wid * _BPW
    obase = wid * _BPW
    pltpu.sync_copy(idx_hbm, idx_v)
    idxv = [idx_v[pl.ds(16 * k, 16)] for k in range(_NG)]

    def compute(in_buf, out_buf):
        @plsc.parallel_loop(0, _R, unroll=8)
        def _(r):
            row = jnp.full((16,), r, jnp.int32)
            for k in range(_NG):
                v = plsc.load_gather(in_buf, [row, idxv[k]])
                out_buf[r, pl.ds(16 * k, 16)] = v

    pltpu.async_copy(x_hbm.at[base], in_bufs[0], in_sems[0])

    def pair_body(i, carry):
        for b in range(2):
            c = 2 * i + b
            nxt = c + 1

            @pl.when(nxt < _BPW)
            def _():
                pltpu.async_copy(x_hbm.at[base + nxt], in_bufs[1 - b], in_sems[1 - b])
            pltpu.make_async_copy(x_hbm.at[base + c], in_bufs[b], in_sems[b]).wait()

            @pl.when(c >= 2)
            def _():
                pltpu.make_async_copy(
                    out_bufs[b], out_hbm.at[obase + c - 2], out_sems[b]).wait()
            compute(in_bufs[b], out_bufs[b])
            pltpu.async_copy(out_bufs[b], out_hbm.at[obase + c], out_sems[b])
        return carry

    lax.fori_loop(0, _BPW // 2, pair_body, 0)
    pltpu.make_async_copy(out_bufs[0], out_hbm.at[obase + _BPW - 2], out_sems[0]).wait()
    pltpu.make_async_copy(out_bufs[1], out_hbm.at[obase + _BPW - 1], out_sems[1]).wait()


def _sc_tail(x, indices):
    sc_call = pl.kernel(
        _sc_body,
        out_type=jax.ShapeDtypeStruct((_SC_B, _R, _D_OUT), jnp.float32),
        mesh=plsc.VectorSubcoreMesh(core_axis_name="c", subcore_axis_name="s"),
        scratch_types=[
            pltpu.VMEM((_D_OUT,), jnp.int32),
            [pltpu.VMEM((_R, _D_IN), jnp.float32) for _ in range(2)],
            [pltpu.VMEM((_R, _D_OUT), jnp.float32) for _ in range(2)],
            [pltpu.SemaphoreType.DMA for _ in range(2)],
            [pltpu.SemaphoreType.DMA for _ in range(2)],
        ],
        compiler_params=pltpu.CompilerParams(needs_layout_passes=False),
    )
    return sc_call(x, indices)


def _tc_body(x_ref, idx_ref, out_ref):
    x = x_ref[...]
    idx = jnp.broadcast_to(idx_ref[...], (_BM, _R, _D_OUT))
    out_ref[...] = jnp.take_along_axis(x, idx, axis=-1)


def _tc_head(x, indices):
    grid = (_SPLIT // _BM,)
    return pl.pallas_call(
        _tc_body,
        grid=grid,
        in_specs=[pl.BlockSpec((_BM, _R, _D_IN), lambda i: (i, 0, 0)),
                  pl.BlockSpec((1, 1, _D_OUT), lambda i: (0, 0, 0))],
        out_specs=pl.BlockSpec((_BM, _R, _D_OUT), lambda i: (i, 0, 0)),
        out_shape=jax.ShapeDtypeStruct((_B, _R, _D_OUT), jnp.float32),
    )(x, indices.reshape(1, 1, _D_OUT))


def kernel(x, indices):
    sc_out = _sc_tail(x, indices)
    tc_out = _tc_head(x, indices)
    return lax.dynamic_update_slice(tc_out, sc_out, (_SPLIT, 0, 0))


# speedup vs baseline: 1.2085x; 1.0029x over previous
import jax
import jax.numpy as jnp
from jax import lax
from jax.experimental import pallas as pl
from jax.experimental.pallas import tpu as pltpu
from jax.experimental.pallas import tpu_sc as plsc

_B, _R, _D_IN, _D_OUT = 4096, 200, 128, 64
_SPLIT = 3072
_NC, _NS = 2, 16
_NW = _NC * _NS
_SC_B = _B - _SPLIT
_BPW = _SC_B // _NW
_NG = _D_OUT // 16
_BM = 32
_BS = 32


def _sc_body(x_hbm, idx_hbm, out_hbm, idx_v, in_bufs, out_bufs, in_sems, out_sems):
    wid = lax.axis_index("s") * _NC + lax.axis_index("c")
    base = _SPLIT + wid * _BPW
    obase = wid * _BPW
    pltpu.sync_copy(idx_hbm, idx_v)
    idxv = [idx_v[pl.ds(16 * k, 16)] for k in range(_NG)]

    def compute(in_buf, out_buf):
        @plsc.parallel_loop(0, _R, unroll=8)
        def _(r):
            row = jnp.full((16,), r, jnp.int32)
            for k in range(_NG):
                v = plsc.load_gather(in_buf, [row, idxv[k]])
                out_buf[r, pl.ds(16 * k, 16)] = v

    pltpu.async_copy(x_hbm.at[base], in_bufs[0], in_sems[0])

    def pair_body(i, carry):
        for b in range(2):
            c = 2 * i + b
            nxt = c + 1

            @pl.when(nxt < _BPW)
            def _():
                pltpu.async_copy(x_hbm.at[base + nxt], in_bufs[1 - b], in_sems[1 - b])
            pltpu.make_async_copy(x_hbm.at[base + c], in_bufs[b], in_sems[b]).wait()

            @pl.when(c >= 2)
            def _():
                pltpu.make_async_copy(
                    out_bufs[b], out_hbm.at[obase + c - 2], out_sems[b]).wait()
            compute(in_bufs[b], out_bufs[b])
            pltpu.async_copy(out_bufs[b], out_hbm.at[obase + c], out_sems[b])
        return carry

    lax.fori_loop(0, _BPW // 2, pair_body, 0)
    pltpu.make_async_copy(out_bufs[0], out_hbm.at[obase + _BPW - 2], out_sems[0]).wait()
    pltpu.make_async_copy(out_bufs[1], out_hbm.at[obase + _BPW - 1], out_sems[1]).wait()


def _sc_tail(x, indices):
    sc_call = pl.kernel(
        _sc_body,
        out_type=jax.ShapeDtypeStruct((_SC_B, _R, _D_OUT), jnp.float32),
        mesh=plsc.VectorSubcoreMesh(core_axis_name="c", subcore_axis_name="s"),
        scratch_types=[
            pltpu.VMEM((_D_OUT,), jnp.int32),
            [pltpu.VMEM((_R, _D_IN), jnp.float32) for _ in range(2)],
            [pltpu.VMEM((_R, _D_OUT), jnp.float32) for _ in range(2)],
            [pltpu.SemaphoreType.DMA for _ in range(2)],
            [pltpu.SemaphoreType.DMA for _ in range(2)],
        ],
        compiler_params=pltpu.CompilerParams(needs_layout_passes=False),
    )
    return sc_call(x, indices)


def _tc_body(x_ref, idx_ref, out_ref):
    x = x_ref[...]
    idx = jnp.broadcast_to(idx_ref[...], (_BM, _R, _D_OUT))
    out_ref[...] = jnp.take_along_axis(x, idx, axis=-1)


def _tc_head(x, indices):
    grid = (_SPLIT // _BM,)
    return pl.pallas_call(
        _tc_body,
        grid=grid,
        in_specs=[pl.BlockSpec((_BM, _R, _D_IN), lambda i: (i, 0, 0)),
                  pl.BlockSpec((1, 1, _D_OUT), lambda i: (0, 0, 0))],
        out_specs=pl.BlockSpec((_BM, _R, _D_OUT), lambda i: (i, 0, 0)),
        out_shape=jax.ShapeDtypeStruct((_B, _R, _D_OUT), jnp.float32),
    )(x, indices.reshape(1, 1, _D_OUT))


def kernel(x, indices):
    sc_out = _sc_tail(x, indices)
    tc_out = _tc_head(x, indices)
    start = lax.optimization_barrier(jnp.int32(_SPLIT))
    return lax.dynamic_update_slice(tc_out, sc_out, (start, 0, 0))


# speedup vs baseline: 1.2102x; 1.0014x over previous
import jax
import jax.numpy as jnp
from jax import lax
from jax.experimental import pallas as pl
from jax.experimental.pallas import tpu as pltpu
from jax.experimental.pallas import tpu_sc as plsc

_B, _R, _D_IN, _D_OUT = 4096, 200, 128, 64
_SPLIT = 3072
_NC, _NS = 2, 16
_NW = _NC * _NS
_SC_B = _B - _SPLIT
_BPW = _SC_B // _NW
_NG = _D_OUT // 16
_BM = 32


def _sc_body(x_hbm, idx_hbm, out_hbm, idx_v, in_bufs, out_bufs, in_sems, out_sems):
    wid = lax.axis_index("s") * _NC + lax.axis_index("c")
    base = _SPLIT + wid * _BPW
    obase = wid * _BPW
    pltpu.sync_copy(idx_hbm, idx_v)
    idxv = [idx_v[pl.ds(16 * k, 16)] for k in range(_NG)]

    def compute(in_buf, out_buf):
        @plsc.parallel_loop(0, _R, unroll=8)
        def _(r):
            row = jnp.full((16,), r, jnp.int32)
            for k in range(_NG):
                v = plsc.load_gather(in_buf, [row, idxv[k]])
                out_buf[r, pl.ds(16 * k, 16)] = v

    pltpu.async_copy(x_hbm.at[base], in_bufs[0], in_sems[0])

    def pair_body(i, carry):
        for b in range(2):
            c = 2 * i + b
            nxt = c + 1

            @pl.when(nxt < _BPW)
            def _():
                pltpu.async_copy(x_hbm.at[base + nxt], in_bufs[1 - b], in_sems[1 - b])
            pltpu.make_async_copy(x_hbm.at[base + c], in_bufs[b], in_sems[b]).wait()

            @pl.when(c >= 2)
            def _():
                pltpu.make_async_copy(
                    out_bufs[b], out_hbm.at[obase + c - 2], out_sems[b]).wait()
            compute(in_bufs[b], out_bufs[b])
            pltpu.async_copy(out_bufs[b], out_hbm.at[obase + c], out_sems[b])
        return carry

    lax.fori_loop(0, _BPW // 2, pair_body, 0)
    pltpu.make_async_copy(out_bufs[0], out_hbm.at[obase + _BPW - 2], out_sems[0]).wait()
    pltpu.make_async_copy(out_bufs[1], out_hbm.at[obase + _BPW - 1], out_sems[1]).wait()


def _sc_tail(x, indices):
    sc_call = pl.kernel(
        _sc_body,
        out_type=jax.ShapeDtypeStruct((_SC_B, _R, _D_OUT), jnp.float32),
        mesh=plsc.VectorSubcoreMesh(core_axis_name="c", subcore_axis_name="s"),
        scratch_types=[
            pltpu.VMEM((_D_OUT,), jnp.int32),
            [pltpu.VMEM((_R, _D_IN), jnp.float32) for _ in range(2)],
            [pltpu.VMEM((_R, _D_OUT), jnp.float32) for _ in range(2)],
            [pltpu.SemaphoreType.DMA for _ in range(2)],
            [pltpu.SemaphoreType.DMA for _ in range(2)],
        ],
        compiler_params=pltpu.CompilerParams(needs_layout_passes=False),
    )
    return sc_call(x, indices)


def _tc_body(x_ref, idx_ref, out_ref):
    x = x_ref[...]
    idx = jnp.broadcast_to(idx_ref[...], (_BM, _R, _D_OUT))
    out_ref[...] = jnp.take_along_axis(x, idx, axis=-1)


def _tc_head(x, indices):
    grid = (_SPLIT // _BM,)
    return pl.pallas_call(
        _tc_body,
        grid=grid,
        in_specs=[pl.BlockSpec((_BM, _R, _D_IN), lambda i: (i, 0, 0)),
                  pl.BlockSpec((1, 1, _D_OUT), lambda i: (0, 0, 0))],
        out_specs=pl.BlockSpec((_BM, _R, _D_OUT), lambda i: (i, 0, 0)),
        out_shape=jax.ShapeDtypeStruct((_B, _R, _D_OUT), jnp.float32),
    )(x, indices.reshape(1, 1, _D_OUT))


def kernel(x, indices):
    sc_out = _sc_tail(x, indices)
    tc_out = _tc_head(x, indices)
    return lax.dynamic_update_slice(tc_out, sc_out, (_SPLIT, 0, 0))
